# Initial kernel scaffold; baseline (speedup 1.0000x reference)
import jax
import jax.numpy as jnp
from jax.experimental import pallas as pl


def kernel(scores, ks):
    raise NotImplementedError("write your pallas kernel here")


# speedup vs baseline: 66.2488x; 66.2488x over previous
import functools

import jax
import jax.numpy as jnp
from jax import lax
from jax.experimental import pallas as pl
from jax.experimental.pallas import tpu as pltpu
from jax.experimental.pallas import tpu_sc as plsc

B = 128
N = 32768
L = 16
NV = N // L
NC = 2
NS = 16
NW = NC * NS
ROWS_PER_W = B // NW
HB = 2048
HB3 = 1024


def _key_from_bits(bits):
    neg = lax.shift_right_arithmetic(bits, 31)
    return bits ^ (neg & jnp.int32(0x7FFFFFFF))


def _body(scores_hbm, ks_hbm, out_hbm, rowbuf, hist, ksv, sem):
    wid = lax.axis_index("s") * NC + lax.axis_index("c")

    lane = lax.iota(jnp.int32, L)
    lane_base = lane * jnp.int32(HB)
    ones_i = jnp.ones((L,), jnp.int32)
    zeros_i = jnp.zeros((L,), jnp.int32)
    zeros_f = jnp.zeros((L,), jnp.float32)
    ones_f = jnp.ones((L,), jnp.float32)

    def zero_hist(i, _):
        hist[pl.ds(i * L, L)] = zeros_i
        return 0

    lax.fori_loop(0, (L * HB) // L, zero_hist, 0)

    pltpu.sync_copy(ks_hbm, ksv)

    def hist_pass(shift, mask_shift, mask_val, bucket_mask):
        def step(v, _):
            s = plsc.bitcast(rowbuf[pl.ds(v * L, L)], jnp.int32)
            bkt = lax.shift_right_logical(s, shift) & jnp.int32(bucket_mask)
            if mask_shift is None:
                plsc.addupdate_scatter(hist, [lane_base + bkt], ones_i)
            else:
                sel = lax.shift_right_logical(s, mask_shift) == mask_val
                plsc.addupdate_scatter(hist, [lane_base + bkt], ones_i,
                                       mask=sel)
            return 0

        lax.fori_loop(0, NV, step, 0)

    def scan_level(nbuckets, limit):
        def chunk(c, carry):
            run, cnt_v, clt_v, mst_v = carry
            base = c * L
            m = zeros_i
            for cc in range(L):
                m = m + hist[pl.ds(cc * HB + base, L)]
            for cc in range(L):
                hist[pl.ds(cc * HB + base, L)] = zeros_i
            pc = plsc.cumsum(m)
            cum = pc + run
            le = cum <= limit
            cnt_v = cnt_v + jnp.where(le, ones_i, zeros_i)
            clt_v = clt_v + jnp.where(le, m, zeros_i)
            star = jnp.logical_and(jnp.logical_not(le), (cum - m) <= limit)
            mst_v = mst_v + jnp.where(star, m, zeros_i)
            run = run + jnp.sum(m)
            return run, cnt_v, clt_v, mst_v

        init = (jnp.int32(0), zeros_i, zeros_i, zeros_i)
        _, cnt_v, clt_v, mst_v = lax.fori_loop(0, nbuckets // L, chunk, init)
        return jnp.sum(cnt_v), jnp.sum(clt_v), jnp.sum(mst_v)

    def do_row(r, _):
        row = wid * ROWS_PER_W + r
        pltpu.sync_copy(scores_hbm.at[row], rowbuf)
        k = plsc.load_gather(ksv, [jnp.full((L,), row, jnp.int32)])[0]

        def pass1(v, _):
            bits = plsc.bitcast(rowbuf[pl.ds(v * L, L)], jnp.int32)
            s = _key_from_bits(bits)
            rowbuf[pl.ds(v * L, L)] = plsc.bitcast(s, jnp.float32)
            bkt = lax.shift_right_logical(s, 21) ^ jnp.int32(0x400)
            plsc.addupdate_scatter(hist, [lane_base + bkt], ones_i)
            return 0

        lax.fori_loop(0, NV, pass1, 0)

        t1 = jnp.int32(N)
        b1f, clt1, m1 = scan_level(HB, t1 - k)
        k2 = k - (t1 - clt1 - m1)
        raw1 = b1f ^ jnp.int32(0x400)

        hist_pass(10, 21, raw1, 0x7FF)
        b2, clt2, m2 = scan_level(HB, m1 - k2)
        k3 = k2 - (m1 - clt2 - m2)
        raw2 = (raw1 << 11) | b2

        hist_pass(0, 10, raw2, 0x3FF)
        b3, clt3, m3 = scan_level(HB3, m2 - k3)
        k4 = k3 - (m2 - clt3 - m3)

        thresh = (raw2 << 10) | b3

        def final_fast(_):
            def step(v, __):
                s = plsc.bitcast(rowbuf[pl.ds(v * L, L)], jnp.int32)
                rowbuf[pl.ds(v * L, L)] = jnp.where(s >= thresh, ones_f,
                                                    zeros_f)
                return 0

            lax.fori_loop(0, NV, step, 0)
            return 0

        def final_tie(_):
            def step(v, eqrun):
                s = plsc.bitcast(rowbuf[pl.ds(v * L, L)], jnp.int32)
                gt = s > thresh
                eq = s == thresh
                e = jnp.where(eq, ones_i, zeros_i)
                rank = plsc.cumsum(e) + eqrun
                sel = jnp.logical_or(gt, jnp.logical_and(eq, rank <= k4))
                rowbuf[pl.ds(v * L, L)] = jnp.where(sel, ones_f, zeros_f)
                return eqrun + jnp.sum(e)

            lax.fori_loop(0, NV, step, jnp.int32(0))
            return 0

        lax.cond(k4 == m3, final_fast, final_tie, 0)

        pltpu.sync_copy(rowbuf, out_hbm.at[row])
        return 0

    lax.fori_loop(0, ROWS_PER_W, do_row, 0)


@jax.jit
def kernel(scores, ks):
    mesh = plsc.VectorSubcoreMesh(core_axis_name="c", subcore_axis_name="s",
                                  num_cores=NC, num_subcores=NS)
    run = pl.kernel(
        _body,
        out_type=jax.ShapeDtypeStruct((B, N), jnp.float32),
        mesh=mesh,
        compiler_params=pltpu.CompilerParams(needs_layout_passes=False),
        scratch_types=[
            pltpu.VMEM((N,), jnp.float32),
            pltpu.VMEM((L * HB,), jnp.int32),
            pltpu.VMEM((B,), jnp.int32),
            pltpu.SemaphoreType.DMA,
        ],
    )
    out = run(scores, ks.astype(jnp.int32).reshape(B))
    return out.astype(bool)


# speedup vs baseline: 82.6886x; 1.2482x over previous
import functools

import jax
import jax.numpy as jnp
from jax import lax
from jax.experimental import pallas as pl
from jax.experimental.pallas import tpu as pltpu
from jax.experimental.pallas import tpu_sc as plsc

B = 128
N = 32768
L = 16
NV = N // L
NC = 2
NS = 16
NW = NC * NS
ROWS_PER_W = B // NW
HB = 2048
HB3 = 1024


def _key_from_bits(bits):
    neg = lax.shift_right_arithmetic(bits, 31)
    return bits ^ (neg & jnp.int32(0x7FFFFFFF))


def _body(scores_hbm, ks_hbm, out_hbm, rowbuf, hist, ksv, sem):
    wid = lax.axis_index("s") * NC + lax.axis_index("c")

    lane = lax.iota(jnp.int32, L)
    lane_base = lane * jnp.int32(HB)
    ones_i = jnp.ones((L,), jnp.int32)
    zeros_i = jnp.zeros((L,), jnp.int32)
    zeros_f = jnp.zeros((L,), jnp.float32)
    ones_f = jnp.ones((L,), jnp.float32)

    def zero_hist(i, _):
        hist[pl.ds(i * L, L)] = zeros_i
        return 0

    lax.fori_loop(0, (L * HB) // L, zero_hist, 0, unroll=8)

    pltpu.sync_copy(ks_hbm, ksv)

    def hist_pass(shift, mask_shift, mask_val, bucket_mask):
        def step(v, _):
            s = plsc.bitcast(rowbuf[pl.ds(v * L, L)], jnp.int32)
            bkt = lax.shift_right_logical(s, shift) & jnp.int32(bucket_mask)
            if mask_shift is None:
                plsc.addupdate_scatter(hist, [lane_base + bkt], ones_i)
            else:
                sel = lax.shift_right_logical(s, mask_shift) == mask_val
                plsc.addupdate_scatter(hist, [lane_base + bkt], ones_i,
                                       mask=sel)
            return 0

        lax.fori_loop(0, NV, step, 0, unroll=8)

    def scan_level(nbuckets, limit):
        def chunk(c, carry):
            run, cnt_v, clt_v, mst_v = carry
            base = c * L
            m = zeros_i
            for cc in range(L):
                m = m + hist[pl.ds(cc * HB + base, L)]
            for cc in range(L):
                hist[pl.ds(cc * HB + base, L)] = zeros_i
            pc = plsc.cumsum(m)
            cum = pc + run
            le = cum <= limit
            cnt_v = cnt_v + jnp.where(le, ones_i, zeros_i)
            clt_v = clt_v + jnp.where(le, m, zeros_i)
            star = jnp.logical_and(jnp.logical_not(le), (cum - m) <= limit)
            mst_v = mst_v + jnp.where(star, m, zeros_i)
            run = run + jnp.sum(m)
            return run, cnt_v, clt_v, mst_v

        init = (jnp.int32(0), zeros_i, zeros_i, zeros_i)
        _, cnt_v, clt_v, mst_v = lax.fori_loop(0, nbuckets // L, chunk, init)
        return jnp.sum(cnt_v), jnp.sum(clt_v), jnp.sum(mst_v)

    def do_row(r, _):
        row = wid * ROWS_PER_W + r
        pltpu.sync_copy(scores_hbm.at[row], rowbuf)
        k = plsc.load_gather(ksv, [jnp.full((L,), row, jnp.int32)])[0]

        def pass1(v, _):
            bits = plsc.bitcast(rowbuf[pl.ds(v * L, L)], jnp.int32)
            s = _key_from_bits(bits)
            rowbuf[pl.ds(v * L, L)] = plsc.bitcast(s, jnp.float32)
            bkt = lax.shift_right_logical(s, 21) ^ jnp.int32(0x400)
            plsc.addupdate_scatter(hist, [lane_base + bkt], ones_i)
            return 0

        lax.fori_loop(0, NV, pass1, 0, unroll=8)

        t1 = jnp.int32(N)
        b1f, clt1, m1 = scan_level(HB, t1 - k)
        k2 = k - (t1 - clt1 - m1)
        raw1 = b1f ^ jnp.int32(0x400)

        hist_pass(10, 21, raw1, 0x7FF)
        b2, clt2, m2 = scan_level(HB, m1 - k2)
        k3 = k2 - (m1 - clt2 - m2)
        raw2 = (raw1 << 11) | b2

        hist_pass(0, 10, raw2, 0x3FF)
        b3, clt3, m3 = scan_level(HB3, m2 - k3)
        k4 = k3 - (m2 - clt3 - m3)

        thresh = (raw2 << 10) | b3

        def final_fast(_):
            def step(v, __):
                s = plsc.bitcast(rowbuf[pl.ds(v * L, L)], jnp.int32)
                rowbuf[pl.ds(v * L, L)] = jnp.where(s >= thresh, ones_f,
                                                    zeros_f)
                return 0

            lax.fori_loop(0, NV, step, 0, unroll=8)
            return 0

        def final_tie(_):
            def step(v, eqrun):
                s = plsc.bitcast(rowbuf[pl.ds(v * L, L)], jnp.int32)
                gt = s > thresh
                eq = s == thresh
                e = jnp.where(eq, ones_i, zeros_i)
                rank = plsc.cumsum(e) + eqrun
                sel = jnp.logical_or(gt, jnp.logical_and(eq, rank <= k4))
                rowbuf[pl.ds(v * L, L)] = jnp.where(sel, ones_f, zeros_f)
                return eqrun + jnp.sum(e)

            lax.fori_loop(0, NV, step, jnp.int32(0), unroll=8)
            return 0

        lax.cond(k4 == m3, final_fast, final_tie, 0)

        pltpu.sync_copy(rowbuf, out_hbm.at[row])
        return 0

    lax.fori_loop(0, ROWS_PER_W, do_row, 0)


@jax.jit
def kernel(scores, ks):
    mesh = plsc.VectorSubcoreMesh(core_axis_name="c", subcore_axis_name="s",
                                  num_cores=NC, num_subcores=NS)
    run = pl.kernel(
        _body,
        out_type=jax.ShapeDtypeStruct((B, N), jnp.float32),
        mesh=mesh,
        compiler_params=pltpu.CompilerParams(needs_layout_passes=False),
        scratch_types=[
            pltpu.VMEM((N,), jnp.float32),
            pltpu.VMEM((L * HB,), jnp.int32),
            pltpu.VMEM((B,), jnp.int32),
            pltpu.SemaphoreType.DMA,
        ],
    )
    out = run(scores, ks.astype(jnp.int32).reshape(B))
    return out.astype(bool)


# speedup vs baseline: 84.9759x; 1.0277x over previous
import functools

import jax
import jax.numpy as jnp
from jax import lax
from jax.experimental import pallas as pl
from jax.experimental.pallas import tpu as pltpu
from jax.experimental.pallas import tpu_sc as plsc

B = 128
N = 32768
L = 16
NV = N // L
NC = 2
NS = 16
NW = NC * NS
ROWS_PER_W = B // NW
HB = 2048
HB3 = 1024
HSTRIDE = HB + 1


def _key_from_bits(bits):
    neg = lax.shift_right_arithmetic(bits, 31)
    return bits ^ (neg & jnp.int32(0x7FFFFFFF))


def _body(scores_hbm, ks_hbm, out_hbm, rowbuf, hist, ksv, sem):
    wid = lax.axis_index("s") * NC + lax.axis_index("c")

    lane = lax.iota(jnp.int32, L)
    lane_base = lane * jnp.int32(HSTRIDE)
    ones_i = jnp.ones((L,), jnp.int32)
    zeros_i = jnp.zeros((L,), jnp.int32)
    zeros_f = jnp.zeros((L,), jnp.float32)
    ones_f = jnp.ones((L,), jnp.float32)

    def zero_hist(i, _):
        hist[pl.ds(i * L, L)] = zeros_i
        return 0

    lax.fori_loop(0, (L * HSTRIDE + L - 1) // L, zero_hist, 0, unroll=8)

    pltpu.sync_copy(ks_hbm, ksv)

    def hist_pass(shift, mask_shift, mask_val, bucket_mask):
        def step(v, _):
            s = plsc.bitcast(rowbuf[pl.ds(v * L, L)], jnp.int32)
            bkt = lax.shift_right_logical(s, shift) & jnp.int32(bucket_mask)
            if mask_shift is None:
                plsc.addupdate_scatter(hist, [lane_base + bkt], ones_i)
            else:
                sel = lax.shift_right_logical(s, mask_shift) == mask_val
                plsc.addupdate_scatter(hist, [lane_base + bkt], ones_i,
                                       mask=sel)
            return 0

        lax.fori_loop(0, NV, step, 0, unroll=8)

    def scan_level(nbuckets, limit):
        def chunk(c, carry):
            run, cnt_v, clt_v, mst_v = carry
            base = c * L
            m = zeros_i
            for cc in range(L):
                m = m + hist[pl.ds(cc * HSTRIDE + base, L)]
            for cc in range(L):
                hist[pl.ds(cc * HSTRIDE + base, L)] = zeros_i
            pc = plsc.cumsum(m)
            cum = pc + run
            le = cum <= limit
            cnt_v = cnt_v + jnp.where(le, ones_i, zeros_i)
            clt_v = clt_v + jnp.where(le, m, zeros_i)
            star = jnp.logical_and(jnp.logical_not(le), (cum - m) <= limit)
            mst_v = mst_v + jnp.where(star, m, zeros_i)
            run = run + jnp.sum(m)
            return run, cnt_v, clt_v, mst_v

        init = (jnp.int32(0), zeros_i, zeros_i, zeros_i)
        _, cnt_v, clt_v, mst_v = lax.fori_loop(0, nbuckets // L, chunk, init)
        return jnp.sum(cnt_v), jnp.sum(clt_v), jnp.sum(mst_v)

    def do_row(r, _):
        row = wid * ROWS_PER_W + r
        pltpu.sync_copy(scores_hbm.at[row], rowbuf)
        k = plsc.load_gather(ksv, [jnp.full((L,), row, jnp.int32)])[0]

        def pass1(v, _):
            bits = plsc.bitcast(rowbuf[pl.ds(v * L, L)], jnp.int32)
            s = _key_from_bits(bits)
            rowbuf[pl.ds(v * L, L)] = plsc.bitcast(s, jnp.float32)
            bkt = lax.shift_right_logical(s, 21) ^ jnp.int32(0x400)
            plsc.addupdate_scatter(hist, [lane_base + bkt], ones_i)
            return 0

        lax.fori_loop(0, NV, pass1, 0, unroll=8)

        t1 = jnp.int32(N)
        b1f, clt1, m1 = scan_level(HB, t1 - k)
        k2 = k - (t1 - clt1 - m1)
        raw1 = b1f ^ jnp.int32(0x400)

        hist_pass(10, 21, raw1, 0x7FF)
        b2, clt2, m2 = scan_level(HB, m1 - k2)
        k3 = k2 - (m1 - clt2 - m2)
        raw2 = (raw1 << 11) | b2

        hist_pass(0, 10, raw2, 0x3FF)
        b3, clt3, m3 = scan_level(HB3, m2 - k3)
        k4 = k3 - (m2 - clt3 - m3)

        thresh = (raw2 << 10) | b3

        def final_fast(_):
            def step(v, __):
                s = plsc.bitcast(rowbuf[pl.ds(v * L, L)], jnp.int32)
                rowbuf[pl.ds(v * L, L)] = jnp.where(s >= thresh, ones_f,
                                                    zeros_f)
                return 0

            lax.fori_loop(0, NV, step, 0, unroll=8)
            return 0

        def final_tie(_):
            def step(v, eqrun):
                s = plsc.bitcast(rowbuf[pl.ds(v * L, L)], jnp.int32)
                gt = s > thresh
                eq = s == thresh
                e = jnp.where(eq, ones_i, zeros_i)
                rank = plsc.cumsum(e) + eqrun
                sel = jnp.logical_or(gt, jnp.logical_and(eq, rank <= k4))
                rowbuf[pl.ds(v * L, L)] = jnp.where(sel, ones_f, zeros_f)
                return eqrun + jnp.sum(e)

            lax.fori_loop(0, NV, step, jnp.int32(0), unroll=8)
            return 0

        lax.cond(k4 == m3, final_fast, final_tie, 0)

        pltpu.sync_copy(rowbuf, out_hbm.at[row])
        return 0

    lax.fori_loop(0, ROWS_PER_W, do_row, 0)


@jax.jit
def kernel(scores, ks):
    mesh = plsc.VectorSubcoreMesh(core_axis_name="c", subcore_axis_name="s",
                                  num_cores=NC, num_subcores=NS)
    run = pl.kernel(
        _body,
        out_type=jax.ShapeDtypeStruct((B, N), jnp.float32),
        mesh=mesh,
        compiler_params=pltpu.CompilerParams(needs_layout_passes=False),
        scratch_types=[
            pltpu.VMEM((N,), jnp.float32),
            pltpu.VMEM((L * HSTRIDE + L,), jnp.int32),
            pltpu.VMEM((B,), jnp.int32),
            pltpu.SemaphoreType.DMA,
        ],
    )
    out = run(scores, ks.astype(jnp.int32).reshape(B))
    return out.astype(bool)
